# baseline (speedup 1.0000x reference)
import jax
import jax.numpy as jnp
from jax import lax
from jax.experimental import pallas as pl
from jax.experimental.pallas import tpu as pltpu

_BT = 256
_NBUF = 4


def _body(actual_ref, nchunks_ref, cb_ref, ct_ref, cl_ref,
          x_ref, o_ref, buf, acc, sems):
    B, T, D = x_ref.shape
    total = nchunks_ref[0]

    def copy_args(g, slot):
        t0 = pl.multiple_of(ct_ref[g], _BT)
        return (x_ref.at[cb_ref[g], pl.ds(t0, _BT), :],
                buf.at[slot], sems.at[slot])

    def issue(g, slot):
        pltpu.make_async_copy(*copy_args(g, slot)).start()

    for k in range(_NBUF):
        @pl.when(k < total)
        def _prime(k=k):
            issue(jnp.int32(k), k)

    def chunk_step(g, carry):
        slot = lax.rem(g, _NBUF)
        pltpu.make_async_copy(*copy_args(g, slot)).wait()
        b = cb_ref[g]
        t0 = ct_ref[g]
        n = actual_ref[b]
        first = t0 == 0
        interior = t0 + _BT <= n

        @pl.when(interior)
        def _plain():
            partial = jnp.sum(buf[slot], axis=0, keepdims=True)
            acc[...] = jnp.where(first, partial, acc[...] + partial)

        @pl.when(jnp.logical_not(interior))
        def _masked():
            row = lax.broadcasted_iota(jnp.int32, (_BT, 1), 0) + t0
            partial = jnp.sum(jnp.where(row < n, buf[slot], 0.0),
                              axis=0, keepdims=True)
            acc[...] = jnp.where(first, partial, acc[...] + partial)

        @pl.when(g + _NBUF < total)
        def _next():
            issue(g + _NBUF, slot)

        @pl.when(cl_ref[g] == 1)
        def _flush():
            o_ref[pl.ds(b, 1), 0, :] = acc[...] / n.astype(jnp.float32)

        return carry

    lax.fori_loop(0, total, chunk_step, 0)


def kernel(x, lens):
    B, T, D = x.shape
    nt = T // _BT
    actual = jnp.floor(lens * T).astype(jnp.int32)

    nbs = jnp.maximum((actual + _BT - 1) // _BT, 1)
    cum = jnp.concatenate([jnp.zeros((1,), jnp.int32),
                           jnp.cumsum(nbs)]).astype(jnp.int32)
    gi = jnp.arange(B * nt, dtype=jnp.int32)
    cb = jnp.clip(jnp.searchsorted(cum, gi, side="right") - 1, 0, B - 1)
    cb = cb.astype(jnp.int32)
    ct = (gi - cum[cb]) * _BT
    cl = (gi == cum[cb + 1] - 1).astype(jnp.int32)
    nchunks = cum[-1:]

    grid_spec = pltpu.PrefetchScalarGridSpec(
        num_scalar_prefetch=5,
        grid=(1,),
        in_specs=[pl.BlockSpec(memory_space=pl.ANY)],
        out_specs=pl.BlockSpec((B, 1, D), lambda i, *_: (0, 0, 0)),
        scratch_shapes=[
            pltpu.VMEM((_NBUF, _BT, D), jnp.float32),
            pltpu.VMEM((1, D), jnp.float32),
            pltpu.SemaphoreType.DMA((_NBUF,)),
        ],
    )
    out = pl.pallas_call(
        _body,
        grid_spec=grid_spec,
        out_shape=jax.ShapeDtypeStruct((B, 1, D), jnp.float32),
    )(actual, nchunks, cb, ct, cl, x)
    return out.reshape(B, D)


# speedup vs baseline: 4.0764x; 4.0764x over previous
import jax
import jax.numpy as jnp
from jax import lax
from jax.experimental import pallas as pl
from jax.experimental.pallas import tpu as pltpu

_BT = 256
_NBUF = 4


def _body(actual_ref, x_ref, o_ref, buf, acc, scb, sct, scl, sems):
    B, T, D = x_ref.shape

    nbs, cums = [], [jnp.int32(0)]
    for j in range(B):
        nb = jnp.maximum((actual_ref[j] + _BT - 1) // _BT, 1)
        nbs.append(nb)
        cums.append(cums[-1] + nb)
    total = cums[-1]

    for j in range(B):
        def prep(i, carry, j=j):
            g = cums[j] + i
            scb[g] = jnp.int32(j)
            sct[g] = i * _BT
            scl[g] = (i == nbs[j] - 1).astype(jnp.int32)
            return carry
        lax.fori_loop(0, nbs[j], prep, 0)

    def copy_args(g, slot):
        t0 = pl.multiple_of(sct[g], _BT)
        return (x_ref.at[scb[g], pl.ds(t0, _BT), :],
                buf.at[slot], sems.at[slot])

    def issue(g, slot):
        pltpu.make_async_copy(*copy_args(g, slot)).start()

    for k in range(_NBUF):
        @pl.when(k < total)
        def _prime(k=k):
            issue(jnp.int32(k), k)

    def chunk_step(g, carry):
        slot = lax.rem(g, _NBUF)
        pltpu.make_async_copy(*copy_args(g, slot)).wait()
        b = scb[g]
        t0 = sct[g]
        n = actual_ref[b]
        first = t0 == 0
        interior = t0 + _BT <= n

        @pl.when(interior)
        def _plain():
            partial = jnp.sum(buf[slot], axis=0, keepdims=True)
            acc[...] = jnp.where(first, partial, acc[...] + partial)

        @pl.when(jnp.logical_not(interior))
        def _masked():
            row = lax.broadcasted_iota(jnp.int32, (_BT, 1), 0) + t0
            partial = jnp.sum(jnp.where(row < n, buf[slot], 0.0),
                              axis=0, keepdims=True)
            acc[...] = jnp.where(first, partial, acc[...] + partial)

        @pl.when(g + _NBUF < total)
        def _next():
            issue(g + _NBUF, slot)

        @pl.when(scl[g] == 1)
        def _flush():
            o_ref[pl.ds(b, 1), 0, :] = acc[...] / n.astype(jnp.float32)

        return carry

    lax.fori_loop(0, total, chunk_step, 0)


def kernel(x, lens):
    B, T, D = x.shape
    nt = T // _BT
    actual = jnp.floor(lens * T).astype(jnp.int32)

    grid_spec = pltpu.PrefetchScalarGridSpec(
        num_scalar_prefetch=1,
        grid=(1,),
        in_specs=[pl.BlockSpec(memory_space=pl.ANY)],
        out_specs=pl.BlockSpec((B, 1, D), lambda i, *_: (0, 0, 0)),
        scratch_shapes=[
            pltpu.VMEM((_NBUF, _BT, D), jnp.float32),
            pltpu.VMEM((1, D), jnp.float32),
            pltpu.SMEM((B * nt,), jnp.int32),
            pltpu.SMEM((B * nt,), jnp.int32),
            pltpu.SMEM((B * nt,), jnp.int32),
            pltpu.SemaphoreType.DMA((_NBUF,)),
        ],
    )
    out = pl.pallas_call(
        _body,
        grid_spec=grid_spec,
        out_shape=jax.ShapeDtypeStruct((B, 1, D), jnp.float32),
    )(actual, x)
    return out.reshape(B, D)


# speedup vs baseline: 4.5145x; 1.1075x over previous
import jax
import jax.numpy as jnp
from jax import lax
from jax.experimental import pallas as pl
from jax.experimental.pallas import tpu as pltpu

_BT = 256
_NBUF = 8


def _body(actual_ref, x_ref, o_ref, buf, acc, scb, sct, scl, sems):
    B, T, D = x_ref.shape

    nbs, cums = [], [jnp.int32(0)]
    for j in range(B):
        nb = jnp.maximum((actual_ref[j] + _BT - 1) // _BT, 1)
        nbs.append(nb)
        cums.append(cums[-1] + nb)
    total = cums[-1]

    for j in range(B):
        def prep(i, carry, j=j):
            g = cums[j] + i
            scb[g] = jnp.int32(j)
            sct[g] = i * _BT
            scl[g] = (i == nbs[j] - 1).astype(jnp.int32)
            return carry
        lax.fori_loop(0, nbs[j], prep, 0)

    def copy_args(g, slot):
        t0 = pl.multiple_of(sct[g], _BT)
        return (x_ref.at[scb[g], pl.ds(t0, _BT), :],
                buf.at[slot], sems.at[slot])

    def issue(g, slot):
        pltpu.make_async_copy(*copy_args(g, slot)).start()

    for k in range(_NBUF):
        @pl.when(k < total)
        def _prime(k=k):
            issue(jnp.int32(k), k)

    def chunk_step(g, carry):
        slot = lax.rem(g, _NBUF)
        pltpu.make_async_copy(*copy_args(g, slot)).wait()
        b = scb[g]
        t0 = sct[g]
        n = actual_ref[b]
        first = t0 == 0
        interior = t0 + _BT <= n

        @pl.when(interior)
        def _plain():
            partial = jnp.sum(buf[slot], axis=0, keepdims=True)
            acc[...] = jnp.where(first, partial, acc[...] + partial)

        @pl.when(jnp.logical_not(interior))
        def _masked():
            row = lax.broadcasted_iota(jnp.int32, (_BT, 1), 0) + t0
            partial = jnp.sum(jnp.where(row < n, buf[slot], 0.0),
                              axis=0, keepdims=True)
            acc[...] = jnp.where(first, partial, acc[...] + partial)

        @pl.when(g + _NBUF < total)
        def _next():
            issue(g + _NBUF, slot)

        @pl.when(scl[g] == 1)
        def _flush():
            o_ref[pl.ds(b, 1), 0, :] = acc[...] / n.astype(jnp.float32)

        return carry

    lax.fori_loop(0, total, chunk_step, 0)


def kernel(x, lens):
    B, T, D = x.shape
    nt = T // _BT
    actual = jnp.floor(lens * T).astype(jnp.int32)

    grid_spec = pltpu.PrefetchScalarGridSpec(
        num_scalar_prefetch=1,
        grid=(1,),
        in_specs=[pl.BlockSpec(memory_space=pl.ANY)],
        out_specs=pl.BlockSpec((B, 1, D), lambda i, *_: (0, 0, 0)),
        scratch_shapes=[
            pltpu.VMEM((_NBUF, _BT, D), jnp.float32),
            pltpu.VMEM((1, D), jnp.float32),
            pltpu.SMEM((B * nt,), jnp.int32),
            pltpu.SMEM((B * nt,), jnp.int32),
            pltpu.SMEM((B * nt,), jnp.int32),
            pltpu.SemaphoreType.DMA((_NBUF,)),
        ],
    )
    out = pl.pallas_call(
        _body,
        grid_spec=grid_spec,
        out_shape=jax.ShapeDtypeStruct((B, 1, D), jnp.float32),
    )(actual, x)
    return out.reshape(B, D)
